# baseline (speedup 1.0000x reference)
import functools

import jax
import jax.numpy as jnp
from jax import lax
from jax.experimental import pallas as pl
from jax.experimental.pallas import tpu as pltpu

_NCPAD = 128
_BI = 400
_SLOTS = 2


def _dot(a, b):
    return lax.dot_general(a, b, (((1,), (0,)), ((), ())),
                           preferred_element_type=jnp.float32)


def _body(nclass, nsteps, nslots, xb_ref, adj_ref, w1_ref, b1_ref, w4_ref,
          b4_ref, o_ref, gb_scr, slab_scr):
    i = pl.program_id(1)
    phase0 = pl.program_id(0) == 0
    hit = i >= nsteps - nslots
    tslot = jnp.where(hit, i - (nsteps - nslots), nslots)

    @pl.when(phase0 | jnp.logical_not(hit))
    def _():
        slab_scr[tslot] = adj_ref[...].astype(jnp.bfloat16)

    @pl.when(phase0)
    def _():
        t = _dot(slab_scr[tslot], xb_ref[...]).astype(jnp.bfloat16)
        h = jnp.maximum(_dot(t, w1_ref[...]) + b1_ref[...], 0.0)
        gb_scr[i] = _dot(h.astype(jnp.bfloat16), w4_ref[...]).astype(jnp.bfloat16)

    @pl.when(jnp.logical_not(phase0))
    def _():
        gb = gb_scr[...].reshape(nsteps * _BI, _NCPAD)
        z = _dot(slab_scr[tslot], gb) + b4_ref[...]
        col = lax.broadcasted_iota(jnp.int32, z.shape, 1)
        zm = jnp.where(col < nclass, z, -jnp.inf)
        m = jnp.max(zm, axis=1, keepdims=True)
        lse = jnp.log(jnp.sum(jnp.exp(zm - m), axis=1, keepdims=True))
        o_ref[...] = lax.slice((z - m) - lse, (0, 0), (_BI, nclass))


@jax.jit
def kernel(x, adj, W1, b1, W4, b4):
    n, nfeat = x.shape
    nhid = W1.shape[1]
    nclass = W4.shape[1]
    nsteps = n // _BI
    nslots = min(_SLOTS, nsteps - 1)

    xb = x.astype(jnp.bfloat16)
    w1b = W1.astype(jnp.bfloat16)
    w4b = jnp.pad(W4, ((0, 0), (0, _NCPAD - nclass))).astype(jnp.bfloat16)
    b1r = b1.reshape(1, nhid)
    b4r = jnp.pad(b4, (0, _NCPAD - nclass)).reshape(1, _NCPAD)

    def adj_imap(p, i):
        i1 = jnp.minimum(i, nsteps - nslots - 1)
        return (jnp.where(p == 0, i, i1), 0)

    return pl.pallas_call(
        functools.partial(_body, nclass, nsteps, nslots),
        grid=(2, nsteps),
        in_specs=[
            pl.BlockSpec((n, nfeat), lambda p, i: (0, 0)),
            pl.BlockSpec((_BI, n), adj_imap),
            pl.BlockSpec((nfeat, nhid), lambda p, i: (0, 0)),
            pl.BlockSpec((1, nhid), lambda p, i: (0, 0)),
            pl.BlockSpec((nhid, _NCPAD), lambda p, i: (0, 0)),
            pl.BlockSpec((1, _NCPAD), lambda p, i: (0, 0)),
        ],
        out_specs=pl.BlockSpec(
            (_BI, nclass), lambda p, i: (jnp.where(p == 0, 0, i), 0)),
        out_shape=jax.ShapeDtypeStruct((n, nclass), jnp.float32),
        scratch_shapes=[
            pltpu.VMEM((nsteps, _BI, _NCPAD), jnp.bfloat16),
            pltpu.VMEM((nslots + 1, _BI, n), jnp.bfloat16),
        ],
        compiler_params=pltpu.CompilerParams(
            dimension_semantics=("arbitrary", "arbitrary"),
            vmem_limit_bytes=64 * 1024 * 1024),
    )(xb, adj, w1b, b1r, w4b, b4r)


# speedup vs baseline: 1.0096x; 1.0096x over previous
import functools

import jax
import jax.numpy as jnp
from jax import lax
from jax.experimental import pallas as pl
from jax.experimental.pallas import tpu as pltpu

_NCPAD = 128
_BI = 400
_STRIDE = 16
_SLOT_CAP = 2
_CHUNK = 80


def _dot(a, b):
    return lax.dot_general(a, b, (((1,), (0,)), ((), ())),
                           preferred_element_type=jnp.float32,
                           precision=lax.Precision.DEFAULT)


def _body(nclass, nsteps, nslots, x_ref, adj_ref, w1_ref, b1_ref, w4_ref,
          b4_ref, o_ref, g_scr, gb_scr, stash_scr, acc_scr):
    i = pl.program_id(1)
    phase0 = pl.program_id(0) == 0
    hit = i < nslots
    slot = i

    @pl.when(phase0)
    def _():
        @pl.when(hit)
        def _():
            stash_scr[slot] = adj_ref[...].astype(jnp.bfloat16)

        t = _dot(adj_ref[...], x_ref[...])
        h = jnp.maximum(_dot(t, w1_ref[...]) + b1_ref[...], 0.0)
        gi = _dot(h, w4_ref[...])
        g_scr[i] = gi
        gb_scr[i] = gi.astype(jnp.bfloat16)

    @pl.when(jnp.logical_not(phase0) & hit)
    def _():
        gb = gb_scr[...].reshape(nsteps * _BI, _NCPAD)
        acc_scr[...] = _dot(stash_scr[slot], gb)

    @pl.when(jnp.logical_not(phase0) & jnp.logical_not(hit))
    def _():
        g = g_scr[...].reshape(nsteps * _BI, _NCPAD)
        acc_scr[...] = _dot(adj_ref[...], g)

    @pl.when(jnp.logical_not(phase0))
    def _():
        z = acc_scr[...] + b4_ref[...]
        col = lax.broadcasted_iota(jnp.int32, z.shape, 1)
        zm = jnp.where(col < nclass, z, -jnp.inf)
        m = jnp.max(zm, axis=1, keepdims=True)
        lse = jnp.log(jnp.sum(jnp.exp(zm - m), axis=1, keepdims=True))
        o_ref[...] = lax.slice((z - m) - lse, (0, 0), (_BI, nclass))


@jax.jit
def kernel(x, adj, W1, b1, W4, b4):
    n, nfeat = x.shape
    nhid = W1.shape[1]
    nclass = W4.shape[1]
    nsteps = n // _BI
    nslots = max(1, min((nsteps + _STRIDE - 1) // _STRIDE, _SLOT_CAP))

    w4p = jnp.pad(W4, ((0, 0), (0, _NCPAD - nclass)))
    b1r = b1.reshape(1, nhid)
    b4r = jnp.pad(b4, (0, _NCPAD - nclass)).reshape(1, _NCPAD)

    def adj_imap(p, i):
        i1 = jnp.maximum(i, nslots)
        return (jnp.where(p == 0, i, i1), 0)

    return pl.pallas_call(
        functools.partial(_body, nclass, nsteps, nslots),
        grid=(2, nsteps),
        in_specs=[
            pl.BlockSpec((n, nfeat), lambda p, i: (0, 0)),
            pl.BlockSpec((_BI, n), adj_imap),
            pl.BlockSpec((nfeat, nhid), lambda p, i: (0, 0)),
            pl.BlockSpec((1, nhid), lambda p, i: (0, 0)),
            pl.BlockSpec((nhid, _NCPAD), lambda p, i: (0, 0)),
            pl.BlockSpec((1, _NCPAD), lambda p, i: (0, 0)),
        ],
        out_specs=pl.BlockSpec(
            (_BI, nclass), lambda p, i: (jnp.where(p == 0, 0, i), 0)),
        out_shape=jax.ShapeDtypeStruct((n, nclass), jnp.float32),
        scratch_shapes=[
            pltpu.VMEM((nsteps, _BI, _NCPAD), jnp.float32),
            pltpu.VMEM((nsteps, _BI, _NCPAD), jnp.bfloat16),
            pltpu.VMEM((nslots, _BI, n), jnp.bfloat16),
            pltpu.VMEM((_BI, _NCPAD), jnp.float32),
        ],
        compiler_params=pltpu.CompilerParams(
            dimension_semantics=("arbitrary", "arbitrary"),
            vmem_limit_bytes=64 * 1024 * 1024),
    )(x, adj, W1, b1r, w4p, b4r)


# speedup vs baseline: 1.0131x; 1.0035x over previous
import functools

import jax
import jax.numpy as jnp
from jax import lax
from jax.experimental import pallas as pl
from jax.experimental.pallas import tpu as pltpu

_NCPAD = 128
_BI = 400
_STRIDE = 16
_SLOT_CAP = 2
_CHUNK = 80


def _dot(a, b):
    return lax.dot_general(a, b, (((1,), (0,)), ((), ())),
                           preferred_element_type=jnp.float32,
                           precision=lax.Precision.DEFAULT)


def _body(nclass, nsteps, nslots, x_ref, adj_ref, w1_ref, b1_ref, w4_ref,
          b4_ref, o_ref, g_scr, gb_scr, stash_scr, acc_scr):
    i = pl.program_id(1)
    phase0 = pl.program_id(0) == 0
    hit = i >= nsteps - nslots
    slot = i - (nsteps - nslots)

    @pl.when(phase0)
    def _():
        @pl.when(hit)
        def _():
            stash_scr[slot] = adj_ref[...].astype(jnp.bfloat16)

        t = _dot(adj_ref[...], x_ref[...])
        h = jnp.maximum(_dot(t, w1_ref[...]) + b1_ref[...], 0.0)
        gi = _dot(h, w4_ref[...])
        g_scr[i] = gi
        gb_scr[i] = gi.astype(jnp.bfloat16)

    @pl.when(jnp.logical_not(phase0) & hit)
    def _():
        gb = gb_scr[...].reshape(nsteps * _BI, _NCPAD)
        acc_scr[...] = _dot(stash_scr[slot], gb)

    @pl.when(jnp.logical_not(phase0) & jnp.logical_not(hit))
    def _():
        g = g_scr[...].reshape(nsteps * _BI, _NCPAD)
        acc_scr[...] = _dot(adj_ref[...], g)

    @pl.when(jnp.logical_not(phase0))
    def _():
        z = acc_scr[...] + b4_ref[...]
        col = lax.broadcasted_iota(jnp.int32, z.shape, 1)
        zm = jnp.where(col < nclass, z, -jnp.inf)
        m = jnp.max(zm, axis=1, keepdims=True)
        lse = jnp.log(jnp.sum(jnp.exp(zm - m), axis=1, keepdims=True))
        o_ref[...] = lax.slice((z - m) - lse, (0, 0), (_BI, nclass))


@jax.jit
def kernel(x, adj, W1, b1, W4, b4):
    n, nfeat = x.shape
    nhid = W1.shape[1]
    nclass = W4.shape[1]
    nsteps = n // _BI
    nslots = max(1, min((nsteps + _STRIDE - 1) // _STRIDE, _SLOT_CAP))

    w4p = jnp.pad(W4, ((0, 0), (0, _NCPAD - nclass)))
    b1r = b1.reshape(1, nhid)
    b4r = jnp.pad(b4, (0, _NCPAD - nclass)).reshape(1, _NCPAD)

    def adj_imap(p, i):
        i1 = jnp.minimum(i, nsteps - nslots - 1)
        return (jnp.where(p == 0, i, i1), 0)

    return pl.pallas_call(
        functools.partial(_body, nclass, nsteps, nslots),
        grid=(2, nsteps),
        in_specs=[
            pl.BlockSpec((n, nfeat), lambda p, i: (0, 0)),
            pl.BlockSpec((_BI, n), adj_imap),
            pl.BlockSpec((nfeat, nhid), lambda p, i: (0, 0)),
            pl.BlockSpec((1, nhid), lambda p, i: (0, 0)),
            pl.BlockSpec((nhid, _NCPAD), lambda p, i: (0, 0)),
            pl.BlockSpec((1, _NCPAD), lambda p, i: (0, 0)),
        ],
        out_specs=pl.BlockSpec(
            (_BI, nclass), lambda p, i: (jnp.where(p == 0, 0, i), 0)),
        out_shape=jax.ShapeDtypeStruct((n, nclass), jnp.float32),
        scratch_shapes=[
            pltpu.VMEM((nsteps, _BI, _NCPAD), jnp.float32),
            pltpu.VMEM((nsteps, _BI, _NCPAD), jnp.bfloat16),
            pltpu.VMEM((nslots, _BI, n), jnp.bfloat16),
            pltpu.VMEM((_BI, _NCPAD), jnp.float32),
        ],
        compiler_params=pltpu.CompilerParams(
            dimension_semantics=("arbitrary", "arbitrary"),
            vmem_limit_bytes=64 * 1024 * 1024),
    )(x, adj, W1, b1r, w4p, b4r)


# speedup vs baseline: 1.0179x; 1.0047x over previous
import functools

import jax
import jax.numpy as jnp
from jax import lax
from jax.experimental import pallas as pl
from jax.experimental.pallas import tpu as pltpu

_NCPAD = 128
_BI = 400
_STRIDE = 16
_SLOT_CAP = 2


def _dot(a, b):
    return lax.dot_general(a, b, (((1,), (0,)), ((), ())),
                           preferred_element_type=jnp.float32,
                           precision=lax.Precision.DEFAULT)


def _body(nclass, nsteps, nslots, x_ref, adj_ref, w1_ref, b1_ref, w4_ref,
          b4_ref, o_ref, g_scr, gb_scr, stash_scr, acc_scr):
    i = pl.program_id(1)
    phase0 = pl.program_id(0) == 0
    hit = i >= nsteps - nslots
    slot = i - (nsteps - nslots)

    @pl.when(phase0)
    def _():
        @pl.when(hit)
        def _():
            stash_scr[slot] = adj_ref[...].astype(jnp.bfloat16)

        t = _dot(adj_ref[...], x_ref[...])
        h = jnp.maximum(_dot(t, w1_ref[...]) + b1_ref[...], 0.0)
        gi = _dot(h, w4_ref[...])
        g_scr[i] = gi
        gb_scr[i] = gi.astype(jnp.bfloat16)

    @pl.when(jnp.logical_not(phase0) & hit)
    def _():
        gb = gb_scr[...].reshape(nsteps * _BI, _NCPAD)
        acc_scr[...] = _dot(stash_scr[slot], gb)

    @pl.when(jnp.logical_not(phase0) & jnp.logical_not(hit))
    def _():
        g = g_scr[...].reshape(nsteps * _BI, _NCPAD)
        acc_scr[...] = _dot(adj_ref[...], g)

    @pl.when(jnp.logical_not(phase0))
    def _():
        z = acc_scr[...] + b4_ref[...]
        col = lax.broadcasted_iota(jnp.int32, z.shape, 1)
        zm = jnp.where(col < nclass, z, -jnp.inf)
        m = jnp.max(zm, axis=1, keepdims=True)
        lse = jnp.log(jnp.sum(jnp.exp(zm - m), axis=1, keepdims=True))
        o_ref[...] = lax.slice((z - m) - lse, (0, 0), (_BI, nclass))


@jax.jit
def kernel(x, adj, W1, b1, W4, b4):
    n, nfeat = x.shape
    nhid = W1.shape[1]
    nclass = W4.shape[1]
    nsteps = n // _BI
    nslots = max(1, min((nsteps + _STRIDE - 1) // _STRIDE, _SLOT_CAP))

    w4p = jnp.pad(W4, ((0, 0), (0, _NCPAD - nclass)))
    b1r = b1.reshape(1, nhid)
    b4r = jnp.pad(b4, (0, _NCPAD - nclass)).reshape(1, _NCPAD)

    def adj_imap(p, i):
        i1 = jnp.minimum(i, nsteps - nslots - 1)
        return (jnp.where(p == 0, i, i1), 0)

    return pl.pallas_call(
        functools.partial(_body, nclass, nsteps, nslots),
        grid=(2, nsteps),
        in_specs=[
            pl.BlockSpec((n, nfeat), lambda p, i: (0, 0)),
            pl.BlockSpec((_BI, n), adj_imap),
            pl.BlockSpec((nfeat, nhid), lambda p, i: (0, 0)),
            pl.BlockSpec((1, nhid), lambda p, i: (0, 0)),
            pl.BlockSpec((nhid, _NCPAD), lambda p, i: (0, 0)),
            pl.BlockSpec((1, _NCPAD), lambda p, i: (0, 0)),
        ],
        out_specs=pl.BlockSpec(
            (_BI, nclass), lambda p, i: (jnp.where(p == 0, 0, i), 0)),
        out_shape=jax.ShapeDtypeStruct((n, nclass), jnp.float32),
        scratch_shapes=[
            pltpu.VMEM((nsteps, _BI, _NCPAD), jnp.float32),
            pltpu.VMEM((nsteps, _BI, _NCPAD), jnp.bfloat16),
            pltpu.VMEM((nslots, _BI, n), jnp.bfloat16),
            pltpu.VMEM((_BI, _NCPAD), jnp.float32),
        ],
        compiler_params=pltpu.CompilerParams(
            dimension_semantics=("arbitrary", "arbitrary"),
            vmem_limit_bytes=64 * 1024 * 1024),
    )(x, adj, W1, b1r, w4p, b4r)
